# baseline (speedup 1.0000x reference)
import functools

import jax
import jax.numpy as jnp
from jax import lax
from jax.experimental import pallas as pl
from jax.experimental.pallas import tpu as pltpu
from jax.experimental.pallas import tpu_sc as plsc

N = 10000
NP = 10240
E = 160000
EP = 163840
D = 256
HD = 128
NC = 2
NS = 16
CH = 64
NCH = EP // (NS * CH)
RPT = NP // NS

_f32 = jnp.float32


DCH = 128
DNCH = EP // (NS * DCH)


def _deg_body(dst_hbm, ones_hbm, out_hbm, dst_v, ones_v, sem, acc):
    c = lax.axis_index("c")
    s = lax.axis_index("s")
    pltpu.sync_copy(dst_hbm.at[s], dst_v)
    pltpu.sync_copy(ones_hbm.at[pl.ds(0, DCH)], ones_v)
    pltpu.sync_copy(ones_hbm.at[pl.ds(s * RPT, RPT)], acc.at[pl.ds(s * RPT, RPT)])
    plsc.subcore_barrier()

    half = DNCH // NC

    def issue(j, carry):
        pltpu.async_copy(ones_v, acc.at[dst_v.at[c * half + j]], sem, add=True)
        return carry

    def drain(j, carry):
        pltpu.make_async_copy(ones_v, acc.at[dst_v.at[c * half + j]], sem).wait()
        return carry

    lax.fori_loop(0, half, issue, 0)
    lax.fori_loop(0, half, drain, 0)
    plsc.subcore_barrier()
    pltpu.sync_copy(acc.at[pl.ds(s * RPT, RPT)], out_hbm.at[c, pl.ds(s * RPT, RPT)])


_deg_call = pl.kernel(
    _deg_body,
    out_type=jax.ShapeDtypeStruct((NC, NP, 8), _f32),
    mesh=plsc.VectorSubcoreMesh(core_axis_name="c", subcore_axis_name="s"),
    scratch_types=[
        pltpu.VMEM((DNCH, DCH), jnp.int32),
        pltpu.VMEM((DCH, 8), _f32),
        pltpu.SemaphoreType.DMA,
        pltpu.VMEM_SHARED((NP, 8), _f32),
    ],
)


NBUF = 4
G = 80
NG = NCH // G


def _agg_body(y_hbm, ed_hbm, out_hbm, ed_v,
              m0, m1, m2, m3, gs0, gs1, gs2, gs3, isem, acc):
    msgs = (m0, m1, m2, m3)
    gsems = (gs0, gs1, gs2, gs3)
    c = lax.axis_index("c")
    s = lax.axis_index("s")
    init_src = y_hbm.at[pl.ds(c * NP + s * RPT, RPT)]
    acc_mine = acc.at[pl.ds(s * RPT, RPT)]
    pltpu.async_copy(init_src, acc_mine, isem)

    def gather(j, b):
        pltpu.async_copy(y_hbm.at[ed_v.at[j, pl.ds(CH, CH)]], msgs[b], gsems[b])

    def wait_gather(j, b):
        pltpu.make_async_copy(y_hbm.at[ed_v.at[j, pl.ds(CH, CH)]],
                              msgs[b], gsems[b]).wait()

    def scatter_sync(j, b):
        pltpu.sync_copy(msgs[b], acc.at[ed_v.at[j, pl.ds(0, CH)]], add=True)

    for g in range(NG):
        pltpu.sync_copy(ed_hbm.at[c, s, pl.ds(g * G, G)], ed_v)
        for b in range(NBUF):
            gather(b, b)
        if g == 0:
            pltpu.make_async_copy(init_src, acc_mine, isem).wait()
            plsc.subcore_barrier()

        def body(i, carry):
            for b in range(NBUF):
                j = i * NBUF + b
                wait_gather(j - NBUF, b)
                scatter_sync(j - NBUF, b)
                gather(j, b)
            return carry

        lax.fori_loop(1, G // NBUF, body, 0)
        for b in range(NBUF):
            j = G - NBUF + b
            wait_gather(j, b)
            scatter_sync(j, b)
    plsc.subcore_barrier()
    pltpu.sync_copy(acc_mine, out_hbm.at[c, pl.ds(s * RPT, RPT)])


_agg_call = pl.kernel(
    _agg_body,
    out_type=jax.ShapeDtypeStruct((NC, NP, HD), _f32),
    mesh=plsc.VectorSubcoreMesh(core_axis_name="c", subcore_axis_name="s"),
    scratch_types=[
        pltpu.VMEM((G, 2 * CH), jnp.int32),
        pltpu.VMEM((CH, HD), _f32),
        pltpu.VMEM((CH, HD), _f32),
        pltpu.VMEM((CH, HD), _f32),
        pltpu.VMEM((CH, HD), _f32),
        pltpu.SemaphoreType.DMA,
        pltpu.SemaphoreType.DMA,
        pltpu.SemaphoreType.DMA,
        pltpu.SemaphoreType.DMA,
        pltpu.SemaphoreType.DMA,
        pltpu.VMEM_SHARED((NP, HD), _f32),
    ],
)


BR = 512


def _dinv_of(degp_block):
    deg = degp_block[0, :, :1] + degp_block[1, :, :1] - 1.0
    return lax.rsqrt(deg)


def _tc1_body(x_ref, w_ref, degp_ref, y_ref, dinv_ref):
    dinv = _dinv_of(degp_ref[...])
    xw = jnp.dot(x_ref[...], w_ref[...], preferred_element_type=_f32)
    y = xw * dinv
    y_ref[0] = y[:, :HD]
    y_ref[1] = y[:, HD:]
    dinv_ref[...] = jnp.broadcast_to(dinv, (BR, HD))


def _tc1(x, W1, degp):
    return pl.pallas_call(
        _tc1_body,
        grid=(NP // BR,),
        in_specs=[
            pl.BlockSpec((BR, D), lambda i: (i, 0)),
            pl.BlockSpec((D, D), lambda i: (0, 0)),
            pl.BlockSpec((NC, BR, 8), lambda i: (0, i, 0)),
        ],
        out_specs=[
            pl.BlockSpec((NC, BR, HD), lambda i: (0, i, 0)),
            pl.BlockSpec((BR, HD), lambda i: (i, 0)),
        ],
        out_shape=[
            jax.ShapeDtypeStruct((NC, NP, HD), _f32),
            jax.ShapeDtypeStruct((NP, HD), _f32),
        ],
    )(x, W1, degp)


def _tc2_body(agg_ref, dinv_ref, b_ref, w_ref, y_ref):
    a = agg_ref[...]
    dinv = dinv_ref[...]
    h0 = jax.nn.relu(a[0] * dinv + b_ref[0:1, :HD])
    h1 = jax.nn.relu(a[1] * dinv + b_ref[0:1, HD:])
    h = jnp.concatenate([h0, h1], axis=1)
    y = jnp.dot(h, w_ref[...], preferred_element_type=_f32)
    y_ref[0] = y[:, :HD] * dinv
    y_ref[1] = y[:, HD:] * dinv


def _tc2(agg1, dinvb, b1, W2):
    return pl.pallas_call(
        _tc2_body,
        grid=(NP // BR,),
        in_specs=[
            pl.BlockSpec((NC, BR, HD), lambda i: (0, i, 0)),
            pl.BlockSpec((BR, HD), lambda i: (i, 0)),
            pl.BlockSpec((1, D), lambda i: (0, 0)),
            pl.BlockSpec((D, D), lambda i: (0, 0)),
        ],
        out_specs=pl.BlockSpec((NC, BR, HD), lambda i: (0, i, 0)),
        out_shape=jax.ShapeDtypeStruct((NC, NP, HD), _f32),
    )(agg1, dinvb, b1, W2)


BR3 = 1000


def _tc3_body(agg_ref, dinv_ref, b_ref, out_ref):
    a = agg_ref[...]
    dinv = dinv_ref[...]
    o0 = a[0] * dinv + b_ref[0:1, :HD]
    o1 = a[1] * dinv + b_ref[0:1, HD:]
    m = jnp.maximum(jnp.max(o0, axis=1, keepdims=True),
                    jnp.max(o1, axis=1, keepdims=True))
    s = (jnp.sum(jnp.exp(o0 - m), axis=1, keepdims=True)
         + jnp.sum(jnp.exp(o1 - m), axis=1, keepdims=True))
    lse = jnp.log(s) + m
    out_ref[:, :HD] = o0 - lse
    out_ref[:, HD:] = o1 - lse


def _tc3(agg2, dinvb, b2):
    return pl.pallas_call(
        _tc3_body,
        grid=(N // BR3,),
        in_specs=[
            pl.BlockSpec((NC, BR3, HD), lambda i: (0, i, 0)),
            pl.BlockSpec((BR3, HD), lambda i: (i, 0)),
            pl.BlockSpec((1, D), lambda i: (0, 0)),
        ],
        out_specs=pl.BlockSpec((BR3, D), lambda i: (i, 0)),
        out_shape=jax.ShapeDtypeStruct((N, D), _f32),
    )(agg2, dinvb, b2)


@jax.jit
def kernel(x, edge_index, W1, b1, W2, b2):
    src = edge_index[0].astype(jnp.int32)
    dst = edge_index[1].astype(jnp.int32)
    pad = EP - E
    srcp = jnp.concatenate([src, jnp.zeros((pad,), jnp.int32)])
    dstp = jnp.concatenate([dst, jnp.full((pad,), NP - 1, jnp.int32)])
    dst_t = dstp.reshape(NS, NCH, CH)
    src_t = srcp.reshape(NS, NCH, CH)
    ed = jnp.stack([jnp.concatenate([dst_t, src_t + c * NP], axis=-1)
                    for c in range(NC)])
    x_pad = jnp.pad(x, ((0, NP - N), (0, 0)))
    ones8 = jnp.ones((NP, 8), _f32)

    degp = _deg_call(dstp.reshape(NS, DNCH, DCH), ones8)
    y1, dinvb = _tc1(x_pad, W1, degp)
    agg1 = _agg_call(y1.reshape(NC * NP, HD), ed)
    y2 = _tc2(agg1, dinvb, b1.reshape(1, D), W2)
    agg2 = _agg_call(y2.reshape(NC * NP, HD), ed)
    return _tc3(agg2, dinvb, b2.reshape(1, D))


# speedup vs baseline: 1.1422x; 1.1422x over previous
import jax
import jax.numpy as jnp
from jax import lax
from jax.experimental import pallas as pl
from jax.experimental.pallas import tpu as pltpu
from jax.experimental.pallas import tpu_sc as plsc

N = 10000
NP = 10240
E = 160000
EP = 163840
D = 256
HD = 128
NC = 2
NS = 16
CH = 64
NCH = EP // (NS * CH)
RPT = NP // NS

_f32 = jnp.float32


def _deg_body(dst_hbm, ones_hbm, out_hbm, dst_v, ones_v, sem, acc):
    c = lax.axis_index("c")
    s = lax.axis_index("s")
    pltpu.sync_copy(dst_hbm.at[s], dst_v)
    pltpu.sync_copy(ones_hbm.at[pl.ds(0, CH)], ones_v)
    pltpu.sync_copy(ones_hbm.at[pl.ds(s * RPT, RPT)], acc.at[pl.ds(s * RPT, RPT)])
    plsc.subcore_barrier()

    half = NCH // NC

    def issue(j, carry):
        pltpu.async_copy(ones_v, acc.at[dst_v.at[c * half + j]], sem, add=True)
        return carry

    def drain(j, carry):
        pltpu.make_async_copy(ones_v, acc.at[dst_v.at[c * half + j]], sem).wait()
        return carry

    lax.fori_loop(0, half, issue, 0)
    lax.fori_loop(0, half, drain, 0)
    plsc.subcore_barrier()
    pltpu.sync_copy(acc.at[pl.ds(s * RPT, RPT)], out_hbm.at[c, pl.ds(s * RPT, RPT)])


_deg_call = pl.kernel(
    _deg_body,
    out_type=jax.ShapeDtypeStruct((NC, NP, 8), _f32),
    mesh=plsc.VectorSubcoreMesh(core_axis_name="c", subcore_axis_name="s"),
    scratch_types=[
        pltpu.VMEM((NCH, CH), jnp.int32),
        pltpu.VMEM((CH, 8), _f32),
        pltpu.SemaphoreType.DMA,
        pltpu.VMEM_SHARED((NP, 8), _f32),
    ],
)


NBUF = 4
G = 80
NG = NCH // G


def _agg_body(y_hbm, ed_hbm, out_hbm, ed_v,
              m0, m1, m2, m3, gs0, gs1, gs2, gs3, isem, acc):
    msgs = (m0, m1, m2, m3)
    gsems = (gs0, gs1, gs2, gs3)
    c = lax.axis_index("c")
    s = lax.axis_index("s")
    init_src = y_hbm.at[pl.ds(c * NP + s * RPT, RPT)]
    acc_mine = acc.at[pl.ds(s * RPT, RPT)]
    pltpu.async_copy(init_src, acc_mine, isem)

    def gather(j, b):
        pltpu.async_copy(y_hbm.at[ed_v.at[j, pl.ds(CH, CH)]], msgs[b], gsems[b])

    def wait_gather(j, b):
        pltpu.make_async_copy(y_hbm.at[ed_v.at[j, pl.ds(CH, CH)]],
                              msgs[b], gsems[b]).wait()

    def scatter_sync(j, b):
        pltpu.sync_copy(msgs[b], acc.at[ed_v.at[j, pl.ds(0, CH)]], add=True)

    for g in range(NG):
        pltpu.sync_copy(ed_hbm.at[c, s, pl.ds(g * G, G)], ed_v)
        for b in range(NBUF):
            gather(b, b)
        if g == 0:
            pltpu.make_async_copy(init_src, acc_mine, isem).wait()
            plsc.subcore_barrier()

        def body(i, carry):
            for b in range(NBUF):
                j = i * NBUF + b
                wait_gather(j - NBUF, b)
                scatter_sync(j - NBUF, b)
                gather(j, b)
            return carry

        lax.fori_loop(1, G // NBUF, body, 0)
        for b in range(NBUF):
            j = G - NBUF + b
            wait_gather(j, b)
            scatter_sync(j, b)
    plsc.subcore_barrier()
    pltpu.sync_copy(acc_mine, out_hbm.at[c, pl.ds(s * RPT, RPT)])


_agg_call = pl.kernel(
    _agg_body,
    out_type=jax.ShapeDtypeStruct((NC, NP, HD), _f32),
    mesh=plsc.VectorSubcoreMesh(core_axis_name="c", subcore_axis_name="s"),
    scratch_types=[
        pltpu.VMEM((G, 2 * CH), jnp.int32),
        pltpu.VMEM((CH, HD), _f32),
        pltpu.VMEM((CH, HD), _f32),
        pltpu.VMEM((CH, HD), _f32),
        pltpu.VMEM((CH, HD), _f32),
        pltpu.SemaphoreType.DMA,
        pltpu.SemaphoreType.DMA,
        pltpu.SemaphoreType.DMA,
        pltpu.SemaphoreType.DMA,
        pltpu.SemaphoreType.DMA,
        pltpu.VMEM_SHARED((NP, HD), _f32),
    ],
)


BR = 512


def _dinv_of(degp_block):
    deg = degp_block[0, :, :1] + degp_block[1, :, :1] - 1.0
    return lax.rsqrt(deg)


def _tc1_body(x_ref, w_ref, degp_ref, y_ref, dinv_ref):
    dinv = _dinv_of(degp_ref[...])
    xw = jnp.dot(x_ref[...], w_ref[...], preferred_element_type=_f32)
    y = xw * dinv
    y_ref[0] = y[:, :HD]
    y_ref[1] = y[:, HD:]
    dinv_ref[...] = jnp.broadcast_to(dinv, (BR, HD))


def _tc1(x, W1, degp):
    return pl.pallas_call(
        _tc1_body,
        grid=(NP // BR,),
        in_specs=[
            pl.BlockSpec((BR, D), lambda i: (i, 0)),
            pl.BlockSpec((D, D), lambda i: (0, 0)),
            pl.BlockSpec((NC, BR, 8), lambda i: (0, i, 0)),
        ],
        out_specs=[
            pl.BlockSpec((NC, BR, HD), lambda i: (0, i, 0)),
            pl.BlockSpec((BR, HD), lambda i: (i, 0)),
        ],
        out_shape=[
            jax.ShapeDtypeStruct((NC, NP, HD), _f32),
            jax.ShapeDtypeStruct((NP, HD), _f32),
        ],
    )(x, W1, degp)


def _tc2_body(agg_ref, dinv_ref, b_ref, w_ref, y_ref):
    a = agg_ref[...]
    dinv = dinv_ref[...]
    h0 = jax.nn.relu(a[0] * dinv + b_ref[0:1, :HD])
    h1 = jax.nn.relu(a[1] * dinv + b_ref[0:1, HD:])
    h = jnp.concatenate([h0, h1], axis=1)
    y = jnp.dot(h, w_ref[...], preferred_element_type=_f32)
    y_ref[0] = y[:, :HD] * dinv
    y_ref[1] = y[:, HD:] * dinv


def _tc2(agg1, dinvb, b1, W2):
    return pl.pallas_call(
        _tc2_body,
        grid=(NP // BR,),
        in_specs=[
            pl.BlockSpec((NC, BR, HD), lambda i: (0, i, 0)),
            pl.BlockSpec((BR, HD), lambda i: (i, 0)),
            pl.BlockSpec((1, D), lambda i: (0, 0)),
            pl.BlockSpec((D, D), lambda i: (0, 0)),
        ],
        out_specs=pl.BlockSpec((NC, BR, HD), lambda i: (0, i, 0)),
        out_shape=jax.ShapeDtypeStruct((NC, NP, HD), _f32),
    )(agg1, dinvb, b1, W2)


BR3 = 1000


def _tc3_body(agg_ref, dinv_ref, b_ref, out_ref):
    a = agg_ref[...]
    dinv = dinv_ref[...]
    o0 = a[0] * dinv + b_ref[0:1, :HD]
    o1 = a[1] * dinv + b_ref[0:1, HD:]
    m = jnp.maximum(jnp.max(o0, axis=1, keepdims=True),
                    jnp.max(o1, axis=1, keepdims=True))
    s = (jnp.sum(jnp.exp(o0 - m), axis=1, keepdims=True)
         + jnp.sum(jnp.exp(o1 - m), axis=1, keepdims=True))
    lse = jnp.log(s) + m
    out_ref[:, :HD] = o0 - lse
    out_ref[:, HD:] = o1 - lse


def _tc3(agg2, dinvb, b2):
    return pl.pallas_call(
        _tc3_body,
        grid=(N // BR3,),
        in_specs=[
            pl.BlockSpec((NC, BR3, HD), lambda i: (0, i, 0)),
            pl.BlockSpec((BR3, HD), lambda i: (i, 0)),
            pl.BlockSpec((1, D), lambda i: (0, 0)),
        ],
        out_specs=pl.BlockSpec((BR3, D), lambda i: (i, 0)),
        out_shape=jax.ShapeDtypeStruct((N, D), _f32),
    )(agg2, dinvb, b2)


@jax.jit
def kernel(x, edge_index, W1, b1, W2, b2):
    src = edge_index[0].astype(jnp.int32)
    dst = edge_index[1].astype(jnp.int32)
    pad = EP - E
    srcp = jnp.concatenate([src, jnp.zeros((pad,), jnp.int32)])
    dstp = jnp.concatenate([dst, jnp.full((pad,), NP - 1, jnp.int32)])
    dst_t = dstp.reshape(NS, NCH, CH)
    src_t = srcp.reshape(NS, NCH, CH)
    ed = jnp.stack([jnp.concatenate([dst_t, src_t + c * NP], axis=-1)
                    for c in range(NC)])
    x_pad = jnp.pad(x, ((0, NP - N), (0, 0)))
    ones8 = jnp.ones((NP, 8), _f32)

    degp = _deg_call(dst_t, ones8)
    y1, dinvb = _tc1(x_pad, W1, degp)
    agg1 = _agg_call(y1.reshape(NC * NP, HD), ed)
    y2 = _tc2(agg1, dinvb, b1.reshape(1, D), W2)
    agg2 = _agg_call(y2.reshape(NC * NP, HD), ed)
    return _tc3(agg2, dinvb, b2.reshape(1, D))
